# baseline (speedup 1.0000x reference)
import functools

import jax
import jax.numpy as jnp
from jax import lax
from jax.experimental import pallas as pl
from jax.experimental.pallas import tpu as pltpu
from jax.experimental.pallas import tpu_sc as plsc

B = 8
N = 8192
S = 512
D = 256
GROUP = 4
SHARD = N // GROUP
CHUNKS = SHARD // 16
UNROLL = 8
ROWS = S // GROUP
PUBW = 32


def _fps_body(xyzp, f, xyz_flat_out, f_out,
              x_ref, y_ref, z_ref, xs_ref, ys_ref, zs_ref, dist_ref,
              pub, cons, idxbuf, xyzflat, idxg, fbuf, sh, sem):
    c = lax.axis_index("c")
    s_id = lax.axis_index("s")
    b = c * 4 + s_id // 4
    m = s_id % 4
    g0 = (s_id // 4) * 4
    base = m * SHARD
    iota = lax.iota(jnp.int32, 16)

    pltpu.sync_copy(xyzp.at[pl.ds((b * 3 + 0) * N, N)], x_ref)
    pltpu.sync_copy(xyzp.at[pl.ds((b * 3 + 1) * N, N)], y_ref)
    pltpu.sync_copy(xyzp.at[pl.ds((b * 3 + 2) * N, N)], z_ref)
    pltpu.sync_copy(xyzp.at[pl.ds((b * 3 + 0) * N + base, SHARD)], xs_ref)
    pltpu.sync_copy(xyzp.at[pl.ds((b * 3 + 1) * N + base, SHARD)], ys_ref)
    pltpu.sync_copy(xyzp.at[pl.ds((b * 3 + 2) * N + base, SHARD)], zs_ref)

    big = jnp.full((16,), 1e10, jnp.float32)

    def init_body(i, carry):
        dist_ref[pl.ds(i * 16, 16)] = big
        return carry

    lax.fori_loop(0, CHUNKS, init_body, 0)

    def publish(wbuf, valv, idxfv):
        pub[pl.ds(0, 16)] = valv
        pub[pl.ds(16, 16)] = idxfv
        pltpu.sync_copy(pub, sh.at[pl.ds(wbuf * (16 * PUBW) + s_id * PUBW,
                                         PUBW)])

    val0 = jnp.where(m == 0, jnp.float32(1.0), jnp.float32(-1.0))
    publish(0, jnp.full((16,), val0, jnp.float32),
            jnp.zeros((16,), jnp.float32))
    plsc.subcore_barrier()

    def step(s_step, rbuf, wbuf):
        pltpu.sync_copy(sh.at[pl.ds(rbuf * (16 * PUBW) + g0 * PUBW,
                                    GROUP * PUBW)], cons)
        v = cons[pl.ds(0, 16)]
        pidxf = cons[pl.ds(16, 16)]
        for r in range(1, GROUP):
            o = r * PUBW
            vr = cons[pl.ds(o, 16)]
            mk = vr > v
            v = jnp.where(mk, vr, v)
            pidxf = jnp.where(mk, cons[pl.ds(o + 16, 16)], pidxf)
        gidxv = pidxf.astype(jnp.int32)

        px = plsc.load_gather(x_ref, [gidxv])
        py = plsc.load_gather(y_ref, [gidxv])
        pz = plsc.load_gather(z_ref, [gidxv])

        posv = 3 * s_step + iota
        valrec = jnp.where(iota == 0, px, jnp.where(iota == 1, py, pz))
        plsc.store_scatter(xyzflat, [posv], valrec, mask=iota < 3)
        plsc.store_scatter(idxbuf, [jnp.full((16,), s_step, jnp.int32)],
                           gidxv, mask=iota == 0)

        @plsc.parallel_loop(0, SHARD, 16, unroll=UNROLL,
                            carry=(jnp.full((16,), -1.0, jnp.float32),
                                   jnp.zeros((16,), jnp.int32)))
        def chunk_loop(off, carry):
            rmax, ridx = carry
            xv = xs_ref[pl.ds(off, 16)]
            yv = ys_ref[pl.ds(off, 16)]
            zv = zs_ref[pl.ds(off, 16)]
            dv = dist_ref[pl.ds(off, 16)]
            dx = xv - px
            dy = yv - py
            dz = zv - pz
            d = (dx * dx + dy * dy) + dz * dz
            dn = jnp.minimum(dv, d)
            dist_ref[pl.ds(off, 16)] = dn
            mk = dn > rmax
            rmax = jnp.where(mk, dn, rmax)
            ridx = jnp.where(mk, base + off + iota, ridx)
            return rmax, ridx

        rmax, ridx = chunk_loop

        gmax = jnp.max(rmax)
        gmaxv = jnp.full((16,), gmax, jnp.float32)
        candf = jnp.where(rmax == gmaxv, ridx.astype(jnp.float32),
                          jnp.float32(3e38))
        gidxf = jnp.min(candf)
        publish(wbuf, gmaxv, jnp.full((16,), gidxf, jnp.float32))
        plsc.subcore_barrier()
        plsc.subcore_barrier()
        plsc.subcore_barrier()
        plsc.subcore_barrier()

    def outer_body(i, carry):
        step(2 * i, 0, 1)
        step(2 * i + 1, 1, 0)
        return carry

    lax.fori_loop(0, S // 2, outer_body, 0)

    def idx_copy(j, carry):
        idxg[pl.ds(j * 16, 16)] = idxbuf[pl.ds(m * ROWS + j * 16, 16)] + b * N
        return carry

    lax.fori_loop(0, ROWS // 16, idx_copy, 0)
    pltpu.async_copy(f.at[idxg], fbuf, sem).wait()
    pltpu.sync_copy(fbuf, f_out.at[pl.ds(b * S + m * ROWS, ROWS)])

    @pl.when(m == 0)
    def _():
        pltpu.sync_copy(xyzflat, xyz_flat_out.at[pl.ds(b * S * 3, S * 3)])


@jax.jit
def kernel(xyz, f):
    xyzp = jnp.transpose(xyz, (0, 2, 1)).reshape(B * 3 * N)
    f2d = f.reshape(B * N, D)
    mesh = plsc.VectorSubcoreMesh(core_axis_name="c", subcore_axis_name="s")
    fps = pl.kernel(
        _fps_body,
        out_type=(
            jax.ShapeDtypeStruct((B * S * 3,), jnp.float32),
            jax.ShapeDtypeStruct((B * S, D), jnp.float32),
        ),
        mesh=mesh,
        compiler_params=pltpu.CompilerParams(needs_layout_passes=False),
        scratch_types=[
            pltpu.VMEM((N,), jnp.float32),
            pltpu.VMEM((N,), jnp.float32),
            pltpu.VMEM((N,), jnp.float32),
            pltpu.VMEM((SHARD,), jnp.float32),
            pltpu.VMEM((SHARD,), jnp.float32),
            pltpu.VMEM((SHARD,), jnp.float32),
            pltpu.VMEM((SHARD,), jnp.float32),
            pltpu.VMEM((PUBW,), jnp.float32),
            pltpu.VMEM((GROUP * PUBW,), jnp.float32),
            pltpu.VMEM((S,), jnp.int32),
            pltpu.VMEM((S * 3,), jnp.float32),
            pltpu.VMEM((ROWS,), jnp.int32),
            pltpu.VMEM((ROWS, D), jnp.float32),
            pltpu.VMEM_SHARED((2 * 16 * PUBW,), jnp.float32),
            pltpu.SemaphoreType.DMA,
        ],
    )
    xyz_flat, f_sampled = fps(xyzp, f2d)
    return xyz_flat.reshape(B, S, 3), f_sampled.reshape(B, S, D)


# speedup vs baseline: 1.1908x; 1.1908x over previous
import functools

import jax
import jax.numpy as jnp
from jax import lax
from jax.experimental import pallas as pl
from jax.experimental.pallas import tpu as pltpu
from jax.experimental.pallas import tpu_sc as plsc

B = 8
N = 8192
S = 512
D = 256
GROUP = 4
SHARD = N // GROUP
CHUNKS = SHARD // 16
UNROLL = 8
ROWS = S // GROUP
PUBW = 32


def _fps_body(xyzp, f, xyz_flat_out, f_out,
              x_ref, y_ref, z_ref, xs_ref, ys_ref, zs_ref, dist_ref,
              pub, cons, idxbuf, xyzflat, idxg, fbuf, sh, sem, psem):
    c = lax.axis_index("c")
    s_id = lax.axis_index("s")
    b = c * 4 + s_id // 4
    m = s_id % 4
    g0 = (s_id // 4) * 4
    base = m * SHARD
    iota = lax.iota(jnp.int32, 16)

    pltpu.sync_copy(xyzp.at[pl.ds((b * 3 + 0) * N, N)], x_ref)
    pltpu.sync_copy(xyzp.at[pl.ds((b * 3 + 1) * N, N)], y_ref)
    pltpu.sync_copy(xyzp.at[pl.ds((b * 3 + 2) * N, N)], z_ref)
    pltpu.sync_copy(xyzp.at[pl.ds((b * 3 + 0) * N + base, SHARD)], xs_ref)
    pltpu.sync_copy(xyzp.at[pl.ds((b * 3 + 1) * N + base, SHARD)], ys_ref)
    pltpu.sync_copy(xyzp.at[pl.ds((b * 3 + 2) * N + base, SHARD)], zs_ref)

    big = jnp.full((16,), 1e10, jnp.float32)

    def init_body(i, carry):
        dist_ref[pl.ds(i * 16, 16)] = big
        return carry

    lax.fori_loop(0, CHUNKS, init_body, 0)

    def publish(wbuf, valv, idxfv):
        pub[pl.ds(0, 16)] = valv
        pub[pl.ds(16, 16)] = idxfv
        pltpu.sync_copy(pub, sh.at[pl.ds(wbuf * (16 * PUBW) + s_id * PUBW,
                                         PUBW)])

    val0 = jnp.where(m == 0, jnp.float32(1.0), jnp.float32(-1.0))
    publish(0, jnp.full((16,), val0, jnp.float32),
            jnp.zeros((16,), jnp.float32))
    plsc.subcore_barrier()

    def step(s_step, rbuf, wbuf):
        pltpu.sync_copy(sh.at[pl.ds(rbuf * (16 * PUBW) + g0 * PUBW,
                                    GROUP * PUBW)], cons)
        v = cons[pl.ds(0, 16)]
        pidxf = cons[pl.ds(16, 16)]
        for r in range(1, GROUP):
            o = r * PUBW
            vr = cons[pl.ds(o, 16)]
            mk = vr > v
            v = jnp.where(mk, vr, v)
            pidxf = jnp.where(mk, cons[pl.ds(o + 16, 16)], pidxf)
        gidxv = pidxf.astype(jnp.int32)

        px = plsc.load_gather(x_ref, [gidxv])
        py = plsc.load_gather(y_ref, [gidxv])
        pz = plsc.load_gather(z_ref, [gidxv])

        @plsc.parallel_loop(0, SHARD, 16, unroll=UNROLL,
                            carry=(jnp.full((16,), -1.0, jnp.float32),
                                   jnp.zeros((16,), jnp.int32)))
        def chunk_loop(off, carry):
            rmax, ridx = carry
            xv = xs_ref[pl.ds(off, 16)]
            yv = ys_ref[pl.ds(off, 16)]
            zv = zs_ref[pl.ds(off, 16)]
            dv = dist_ref[pl.ds(off, 16)]
            dx = xv - px
            dy = yv - py
            dz = zv - pz
            d = (dx * dx + dy * dy) + dz * dz
            dn = jnp.minimum(dv, d)
            dist_ref[pl.ds(off, 16)] = dn
            mk = dn > rmax
            rmax = jnp.where(mk, dn, rmax)
            ridx = jnp.where(mk, base + off + iota, ridx)
            return rmax, ridx

        rmax, ridx = chunk_loop

        gmax = jnp.max(rmax)
        gmaxv = jnp.full((16,), gmax, jnp.float32)
        candf = jnp.where(rmax == gmaxv, ridx.astype(jnp.float32),
                          jnp.float32(3e38))
        gidxf = jnp.min(candf)
        pub[pl.ds(0, 16)] = gmaxv
        pub[pl.ds(16, 16)] = jnp.full((16,), gidxf, jnp.float32)
        cp = pltpu.async_copy(
            pub, sh.at[pl.ds(wbuf * (16 * PUBW) + s_id * PUBW, PUBW)], psem)

        posv = 3 * s_step + iota
        valrec = jnp.where(iota == 0, px, jnp.where(iota == 1, py, pz))
        plsc.store_scatter(xyzflat, [posv], valrec, mask=iota < 3)
        plsc.store_scatter(idxbuf, [jnp.full((16,), s_step, jnp.int32)],
                           gidxv, mask=iota == 0)
        cp.wait()
        plsc.subcore_barrier()

    def outer_body(i, carry):
        step(2 * i, 0, 1)
        step(2 * i + 1, 1, 0)
        return carry

    lax.fori_loop(0, S // 2, outer_body, 0)

    def idx_copy(j, carry):
        idxg[pl.ds(j * 16, 16)] = idxbuf[pl.ds(m * ROWS + j * 16, 16)] + b * N
        return carry

    lax.fori_loop(0, ROWS // 16, idx_copy, 0)
    pltpu.async_copy(f.at[idxg], fbuf, sem).wait()
    pltpu.sync_copy(fbuf, f_out.at[pl.ds(b * S + m * ROWS, ROWS)])

    @pl.when(m == 0)
    def _():
        pltpu.sync_copy(xyzflat, xyz_flat_out.at[pl.ds(b * S * 3, S * 3)])


@jax.jit
def kernel(xyz, f):
    xyzp = jnp.transpose(xyz, (0, 2, 1)).reshape(B * 3 * N)
    f2d = f.reshape(B * N, D)
    mesh = plsc.VectorSubcoreMesh(core_axis_name="c", subcore_axis_name="s")
    fps = pl.kernel(
        _fps_body,
        out_type=(
            jax.ShapeDtypeStruct((B * S * 3,), jnp.float32),
            jax.ShapeDtypeStruct((B * S, D), jnp.float32),
        ),
        mesh=mesh,
        compiler_params=pltpu.CompilerParams(needs_layout_passes=False),
        scratch_types=[
            pltpu.VMEM((N,), jnp.float32),
            pltpu.VMEM((N,), jnp.float32),
            pltpu.VMEM((N,), jnp.float32),
            pltpu.VMEM((SHARD,), jnp.float32),
            pltpu.VMEM((SHARD,), jnp.float32),
            pltpu.VMEM((SHARD,), jnp.float32),
            pltpu.VMEM((SHARD,), jnp.float32),
            pltpu.VMEM((PUBW,), jnp.float32),
            pltpu.VMEM((GROUP * PUBW,), jnp.float32),
            pltpu.VMEM((S,), jnp.int32),
            pltpu.VMEM((S * 3,), jnp.float32),
            pltpu.VMEM((ROWS,), jnp.int32),
            pltpu.VMEM((ROWS, D), jnp.float32),
            pltpu.VMEM_SHARED((2 * 16 * PUBW,), jnp.float32),
            pltpu.SemaphoreType.DMA,
            pltpu.SemaphoreType.DMA,
        ],
    )
    xyz_flat, f_sampled = fps(xyzp, f2d)
    return xyz_flat.reshape(B, S, 3), f_sampled.reshape(B, S, D)


# speedup vs baseline: 1.1917x; 1.0008x over previous
import jax
import jax.numpy as jnp
from jax import lax
from jax.experimental import pallas as pl
from jax.experimental.pallas import tpu as pltpu
from jax.experimental.pallas import tpu_sc as plsc

B = 8
N = 8192
S = 512
D = 256
GROUP = 4
SHARD = N // GROUP
CHUNKS = SHARD // 16
UNROLL = 8
ROWS = S // GROUP
PUBW = 32


def _fps_body(xyzp, f, xyz_flat_out, f_out,
              x_ref, y_ref, z_ref, xs_ref, ys_ref, zs_ref, dist_ref,
              pub, cons, idxbuf, xyzflat, idxg, fbuf, sh, sem, psem):
    c = lax.axis_index("c")
    s_id = lax.axis_index("s")
    b = c * 4 + s_id // 4
    m = s_id % 4
    g0 = (s_id // 4) * 4
    base = m * SHARD
    iota = lax.iota(jnp.int32, 16)

    pltpu.sync_copy(xyzp.at[pl.ds((b * 3 + 0) * N, N)], x_ref)
    pltpu.sync_copy(xyzp.at[pl.ds((b * 3 + 1) * N, N)], y_ref)
    pltpu.sync_copy(xyzp.at[pl.ds((b * 3 + 2) * N, N)], z_ref)
    pltpu.sync_copy(xyzp.at[pl.ds((b * 3 + 0) * N + base, SHARD)], xs_ref)
    pltpu.sync_copy(xyzp.at[pl.ds((b * 3 + 1) * N + base, SHARD)], ys_ref)
    pltpu.sync_copy(xyzp.at[pl.ds((b * 3 + 2) * N + base, SHARD)], zs_ref)

    big = jnp.full((16,), 1e10, jnp.float32)

    def init_body(i, carry):
        dist_ref[pl.ds(i * 16, 16)] = big
        return carry

    lax.fori_loop(0, CHUNKS, init_body, 0)

    def publish(wbuf, valv, idxfv):
        pub[pl.ds(0, 16)] = valv
        pub[pl.ds(16, 16)] = idxfv
        pltpu.sync_copy(pub, sh.at[pl.ds(wbuf * (16 * PUBW) + s_id * PUBW,
                                         PUBW)])

    val0 = jnp.where(m == 0, jnp.float32(1.0), jnp.float32(-1.0))
    publish(0, jnp.full((16,), val0, jnp.float32),
            jnp.zeros((16,), jnp.float32))
    plsc.subcore_barrier()

    def step(s_step, rbuf, wbuf):
        pltpu.sync_copy(sh.at[pl.ds(rbuf * (16 * PUBW) + g0 * PUBW,
                                    GROUP * PUBW)], cons)
        v = cons[pl.ds(0, 16)]
        pidxf = cons[pl.ds(16, 16)]
        for r in range(1, GROUP):
            o = r * PUBW
            vr = cons[pl.ds(o, 16)]
            mk = vr > v
            v = jnp.where(mk, vr, v)
            pidxf = jnp.where(mk, cons[pl.ds(o + 16, 16)], pidxf)
        gidxv = pidxf.astype(jnp.int32)

        px = plsc.load_gather(x_ref, [gidxv])
        py = plsc.load_gather(y_ref, [gidxv])
        pz = plsc.load_gather(z_ref, [gidxv])

        @plsc.parallel_loop(0, SHARD, 16, unroll=UNROLL,
                            carry=(jnp.full((16,), -1.0, jnp.float32),
                                   jnp.zeros((16,), jnp.int32)))
        def chunk_loop(off, carry):
            rmax, ridx = carry
            xv = xs_ref[pl.ds(off, 16)]
            yv = ys_ref[pl.ds(off, 16)]
            zv = zs_ref[pl.ds(off, 16)]
            dv = dist_ref[pl.ds(off, 16)]
            dx = xv - px
            dy = yv - py
            dz = zv - pz
            d = (dx * dx + dy * dy) + dz * dz
            dn = jnp.minimum(dv, d)
            dist_ref[pl.ds(off, 16)] = dn
            mk = dn > rmax
            rmax = jnp.where(mk, dn, rmax)
            ridx = jnp.where(mk, base + off + iota, ridx)
            return rmax, ridx

        rmax, ridx = chunk_loop

        gmax = jnp.max(rmax)
        gmaxv = jnp.full((16,), gmax, jnp.float32)
        candf = jnp.where(rmax == gmaxv, ridx.astype(jnp.float32),
                          jnp.float32(3e38))
        gidxf = jnp.min(candf)
        pub[pl.ds(0, 16)] = gmaxv
        pub[pl.ds(16, 16)] = jnp.full((16,), gidxf, jnp.float32)
        cp = pltpu.async_copy(
            pub, sh.at[pl.ds(wbuf * (16 * PUBW) + s_id * PUBW, PUBW)], psem)

        posv = 3 * s_step + iota
        valrec = jnp.where(iota == 0, px, jnp.where(iota == 1, py, pz))
        plsc.store_scatter(xyzflat, [posv], valrec, mask=iota < 3)
        plsc.store_scatter(idxbuf, [jnp.full((16,), s_step, jnp.int32)],
                           gidxv, mask=iota == 0)
        cp.wait()
        plsc.subcore_barrier()

    def outer_body(i, carry):
        step(2 * i, 0, 1)
        step(2 * i + 1, 1, 0)
        return carry

    lax.fori_loop(0, S // 2, outer_body, 0)

    def idx_copy(j, carry):
        idxg[pl.ds(j * 16, 16)] = idxbuf[pl.ds(m * ROWS + j * 16, 16)] + b * N
        return carry

    lax.fori_loop(0, ROWS // 16, idx_copy, 0)
    pltpu.async_copy(f.at[idxg], fbuf, sem).wait()
    pltpu.sync_copy(fbuf, f_out.at[pl.ds(b * S + m * ROWS, ROWS)])

    @pl.when(m == 0)
    def _():
        pltpu.sync_copy(xyzflat, xyz_flat_out.at[pl.ds(b * S * 3, S * 3)])


@jax.jit
def kernel(xyz, f):
    xyzp = jnp.transpose(xyz, (0, 2, 1)).reshape(B * 3 * N)
    f2d = f.reshape(B * N, D)
    mesh = plsc.VectorSubcoreMesh(core_axis_name="c", subcore_axis_name="s")
    fps = pl.kernel(
        _fps_body,
        out_type=(
            jax.ShapeDtypeStruct((B * S * 3,), jnp.float32),
            jax.ShapeDtypeStruct((B * S, D), jnp.float32),
        ),
        mesh=mesh,
        compiler_params=pltpu.CompilerParams(needs_layout_passes=False),
        scratch_types=[
            pltpu.VMEM((N,), jnp.float32),
            pltpu.VMEM((N,), jnp.float32),
            pltpu.VMEM((N,), jnp.float32),
            pltpu.VMEM((SHARD,), jnp.float32),
            pltpu.VMEM((SHARD,), jnp.float32),
            pltpu.VMEM((SHARD,), jnp.float32),
            pltpu.VMEM((SHARD,), jnp.float32),
            pltpu.VMEM((PUBW,), jnp.float32),
            pltpu.VMEM((GROUP * PUBW,), jnp.float32),
            pltpu.VMEM((S,), jnp.int32),
            pltpu.VMEM((S * 3,), jnp.float32),
            pltpu.VMEM((ROWS,), jnp.int32),
            pltpu.VMEM((ROWS, D), jnp.float32),
            pltpu.VMEM_SHARED((2 * 16 * PUBW,), jnp.float32),
            pltpu.SemaphoreType.DMA,
            pltpu.SemaphoreType.DMA,
        ],
    )
    xyz_flat, f_sampled = fps(xyzp, f2d)
    return xyz_flat.reshape(B, S, 3), f_sampled.reshape(B, S, D)
